# Initial kernel scaffold; baseline (speedup 1.0000x reference)
import jax
import jax.numpy as jnp
from jax.experimental import pallas as pl


def kernel(x, edge_index, edge_weight, pos_edge_index, neg_edge_index, W1, a1s, a1d, b1, W2, a2s, a2d, b2, Wl, bl):
    raise NotImplementedError("write your pallas kernel here")


# speedup vs baseline: 8.8501x; 8.8501x over previous
import functools

import jax
import jax.numpy as jnp
from jax import lax
from jax.experimental import pallas as pl
from jax.experimental.pallas import tpu as pltpu
from jax.experimental.pallas import tpu_sc as plsc

N_NODES = 10000
N_EDGES = 160000
F_IN = 256
HEADS = 2
HID = 256

NP = 10240
DUMMY = 10001
LANE = 16
NSUB = 16
NCORE = 2
BT = 128
CW = 64
NBW = 88
E_C = NBW * BT
E_PAD = NSUB * E_C
NBT = E_PAD // BT
STRIPE = NP // NSUB

DEC_ROWS = 1280
DEC_NB = DEC_ROWS // 32
E_DEC_PAD = DEC_ROWS * BT


def _k1_body(x_ref, w_ref, a_ref, *outs):
  h = jnp.dot(x_ref[...], w_ref[...], preferred_element_type=jnp.float32)
  outs[-1][...] = jnp.dot(h, a_ref[...], preferred_element_type=jnp.float32)
  for i, o in enumerate(outs[:-1]):
    o[...] = h[:, i * CW:(i + 1) * CW]


def _tc_layer1(x, W1, A1):
  br = 1000
  grid = (N_NODES // br,)
  hs = [jax.ShapeDtypeStruct((N_NODES, CW), jnp.float32) for _ in range(8)]
  return pl.pallas_call(
      _k1_body,
      grid=grid,
      in_specs=[
          pl.BlockSpec((br, F_IN), lambda i: (i, 0)),
          pl.BlockSpec((F_IN, 2 * HID), lambda i: (0, 0)),
          pl.BlockSpec((2 * HID, 8), lambda i: (0, 0)),
      ],
      out_specs=[pl.BlockSpec((br, CW), lambda i: (i, 0)) for _ in range(8)]
      + [pl.BlockSpec((br, 8), lambda i: (i, 0))],
      out_shape=hs + [jax.ShapeDtypeStruct((N_NODES, 8), jnp.float32)],
  )(x, W1, A1)


def _k2_body(*refs):
  gs, (b_ref, w_ref, a_ref), outs = refs[:8], refs[8:11], refs[11:]
  z = jnp.concatenate([g[...] for g in gs], axis=1)
  z = jnp.maximum(z + b_ref[...], 0.0)
  h = jnp.dot(z, w_ref[...], preferred_element_type=jnp.float32)
  outs[-1][...] = jnp.dot(h, a_ref[...], preferred_element_type=jnp.float32)
  for i, o in enumerate(outs[:-1]):
    o[...] = h[:, i * CW:(i + 1) * CW]


def _tc_layer2(gs, b1, W2, A2):
  br = 1024
  grid = (NP // br,)
  hs = [jax.ShapeDtypeStruct((NP, CW), jnp.float32) for _ in range(4)]
  return pl.pallas_call(
      _k2_body,
      grid=grid,
      in_specs=[pl.BlockSpec((br, CW), lambda i: (i, 0)) for _ in range(8)]
      + [
          pl.BlockSpec((1, 2 * HID), lambda i: (0, 0)),
          pl.BlockSpec((2 * HID, HID), lambda i: (0, 0)),
          pl.BlockSpec((HID, 8), lambda i: (0, 0)),
      ],
      out_specs=[pl.BlockSpec((br, CW), lambda i: (i, 0)) for _ in range(4)]
      + [pl.BlockSpec((br, 8), lambda i: (i, 0))],
      out_shape=hs + [jax.ShapeDtypeStruct((NP, 8), jnp.float32)],
  )(*gs, b1, W2, A2)


def _k3_body(g0, g1, g2, g3, b_ref, wa_ref, wb_ref, bl_ref, p_ref, q_ref):
  z = jnp.concatenate([g0[...], g1[...], g2[...], g3[...]], axis=1) + b_ref[...]
  p_ref[...] = jnp.sum(z * wa_ref[...], axis=1) + bl_ref[0]
  q_ref[...] = jnp.sum(z * wb_ref[...], axis=1)


def _tc_decode_tables(gs, b2, wa, wb, bl):
  br = 1024
  grid = (NP // br,)
  return pl.pallas_call(
      _k3_body,
      grid=grid,
      in_specs=[pl.BlockSpec((br, CW), lambda i: (i, 0)) for _ in range(4)]
      + [
          pl.BlockSpec((1, HID), lambda i: (0, 0)),
          pl.BlockSpec((1, HID), lambda i: (0, 0)),
          pl.BlockSpec((1, HID), lambda i: (0, 0)),
          pl.BlockSpec(memory_space=pltpu.SMEM),
      ],
      out_specs=[pl.BlockSpec((br,), lambda i: (i,)) for _ in range(2)],
      out_shape=[jax.ShapeDtypeStruct((NP,), jnp.float32) for _ in range(2)],
  )(*gs, b2, wa, wb, bl)


def _sc_gat(src2, dst2, asrc, adst, h_chunks):
  nch = len(h_chunks)
  pc = nch // NCORE
  mesh = plsc.VectorSubcoreMesh(core_axis_name="c", subcore_axis_name="s",
                                num_cores=NCORE, num_subcores=NSUB)

  def body(src_hbm, dst_hbm, asrc_hbm, adst_hbm, *rest):
    h_refs = rest[:nch]
    out_refs = rest[nch : 2 * nch]
    (src_v, dst_v, coef_v, asrc_t, adst_t, denom_t, rows_v, zero1_v, sem,
     acc_s, denom_s) = rest[2 * nch :]
    c = lax.axis_index("c")
    s = lax.axis_index("s")
    hd = c if nch == 8 else 0

    pltpu.sync_copy(src_hbm.at[pl.ds(s * NBW, NBW)], src_v)
    pltpu.sync_copy(dst_hbm.at[pl.ds(s * NBW, NBW)], dst_v)
    pltpu.sync_copy(asrc_hbm.at[hd], asrc_t)
    pltpu.sync_copy(adst_hbm.at[hd], adst_t)

    neg = jnp.full((LANE,), -1e30, jnp.float32)

    def _mx(i, carry):
      ms, md = carry
      sl = pl.ds(i * LANE, LANE)
      return jnp.maximum(ms, asrc_t[sl]), jnp.maximum(md, adst_t[sl])

    ms, md = lax.fori_loop(0, NP // LANE, _mx, (neg, neg))
    ms_s, md_s = ms[0], md[0]
    for i in range(1, LANE):
      ms_s = jnp.maximum(ms_s, ms[i])
      md_s = jnp.maximum(md_s, md[i])
    smax = ms_s + md_s
    shift = jnp.where(smax > 0, smax, 0.2 * smax)

    z16 = jnp.zeros((LANE,), jnp.float32)

    @pl.loop(0, STRIPE // LANE)
    def _zz(i):
      zero1_v[pl.ds(i * LANE, LANE)] = z16

    @pl.loop(0, BT)
    def _zr(r):
      for i2 in range(CW // LANE):
        rows_v[r, pl.ds(i2 * LANE, LANE)] = z16

    pltpu.sync_copy(zero1_v, denom_s.at[pl.ds(s * STRIPE, STRIPE)])
    plsc.subcore_barrier()

    @pl.loop(0, NBW)
    def _p1(j):
      for k in range(8):
        sl = pl.ds(k * LANE, LANE)
        sv = src_v[j, sl]
        dv = dst_v[j, sl]
        al = plsc.load_gather(asrc_t, [sv]) + plsc.load_gather(adst_t, [dv])
        al = jnp.where(al > 0, al, 0.2 * al)
        coef_v[j, sl] = jnp.exp(al - shift)
      pltpu.sync_copy(coef_v.at[j], denom_s.at[dst_v.at[j]], add=True)

    plsc.subcore_barrier()

    pltpu.sync_copy(denom_s, denom_t)

    @pl.loop(0, NBW)
    def _p2(j):
      for k in range(8):
        sl = pl.ds(k * LANE, LANE)
        dv = dst_v[j, sl]
        den = plsc.load_gather(denom_t, [dv])
        coef_v[j, sl] = coef_v[j, sl] / (den + 1e-16)

    def agg(tab):
      @pl.loop(0, NBW)
      def _p3(j):
        pltpu.async_copy(tab.at[src_v.at[j]], rows_v, sem).wait()

        @pl.loop(0, BT // LANE)
        def _sc(k):
          cv = coef_v[j, pl.ds(k * LANE, LANE)]
          for rr in range(LANE):
            cf = cv[rr]
            r = k * LANE + rr
            for i2 in range(CW // LANE):
              sl = pl.ds(i2 * LANE, LANE)
              rows_v[r, sl] = rows_v[r, sl] * cf

        pltpu.sync_copy(rows_v, acc_s.at[dst_v.at[j]], add=True)

    for p in range(pc):
      if p > 0:
        @pl.loop(0, BT)
        def _zr2(r):
          for i2 in range(CW // LANE):
            rows_v[r, pl.ds(i2 * LANE, LANE)] = z16

      for zz in range(STRIPE // BT):
        pltpu.sync_copy(rows_v, acc_s.at[pl.ds(s * STRIPE + zz * BT, BT)])
      plsc.subcore_barrier()

      @pl.when(c == 0)
      def _c0():
        agg(h_refs[p])

      @pl.when(c == 1)
      def _c1():
        agg(h_refs[pc + p])

      plsc.subcore_barrier()

      @pl.when(c == 0)
      def _o0():
        pltpu.sync_copy(acc_s.at[pl.ds(s * STRIPE, STRIPE)],
                        out_refs[p].at[pl.ds(s * STRIPE, STRIPE)])

      @pl.when(c == 1)
      def _o1():
        pltpu.sync_copy(acc_s.at[pl.ds(s * STRIPE, STRIPE)],
                        out_refs[pc + p].at[pl.ds(s * STRIPE, STRIPE)])

      if p + 1 < pc:
        plsc.subcore_barrier()

  out_type = [jax.ShapeDtypeStruct((NP, CW), jnp.float32) for _ in range(nch)]
  scratch = [
      pltpu.VMEM((NBW, BT), jnp.int32),
      pltpu.VMEM((NBW, BT), jnp.int32),
      pltpu.VMEM((NBW, BT), jnp.float32),
      pltpu.VMEM((NP,), jnp.float32),
      pltpu.VMEM((NP,), jnp.float32),
      pltpu.VMEM((NP,), jnp.float32),
      pltpu.VMEM((BT, CW), jnp.float32),
      pltpu.VMEM((STRIPE,), jnp.float32),
      pltpu.SemaphoreType.DMA,
      pltpu.VMEM_SHARED((NP, CW), jnp.float32),
      pltpu.VMEM_SHARED((NP,), jnp.float32),
  ]
  fn = pl.kernel(body, out_type=out_type, mesh=mesh, scratch_types=scratch,
                 compiler_params=pltpu.CompilerParams(
                     needs_layout_passes=False, use_tc_tiling_on_sc=False))
  return fn(src2, dst2, asrc, adst, *h_chunks)


def _sc_decode(p_hbm, q_hbm, ps, pd, ns, nd):
  mesh = plsc.VectorSubcoreMesh(core_axis_name="c", subcore_axis_name="s",
                                num_cores=NCORE, num_subcores=NSUB)

  def body(p_in, q_in, ps_in, pd_in, ns_in, nd_in, pos_out, neg_out,
           p_t, q_t, idx_s, idx_d, out_v):
    c = lax.axis_index("c")
    s = lax.axis_index("s")
    w = s * NCORE + c
    pltpu.sync_copy(p_in, p_t)
    pltpu.sync_copy(q_in, q_t)
    for src_h, dst_h, out_h in ((ps_in, pd_in, pos_out),
                                (ns_in, nd_in, neg_out)):
      pltpu.sync_copy(src_h.at[pl.ds(w * DEC_NB, DEC_NB)], idx_s)
      pltpu.sync_copy(dst_h.at[pl.ds(w * DEC_NB, DEC_NB)], idx_d)

      @pl.loop(0, DEC_NB)
      def _dj(j):
        for k in range(8):
          sl = pl.ds(k * LANE, LANE)
          sv = idx_s[j, sl]
          dv = idx_d[j, sl]
          out_v[j, sl] = (plsc.load_gather(p_t, [sv])
                          + plsc.load_gather(q_t, [dv]))

      pltpu.sync_copy(out_v, out_h.at[pl.ds(w * DEC_NB, DEC_NB)])

  out_type = [jax.ShapeDtypeStruct((DEC_ROWS, BT), jnp.float32)
              for _ in range(2)]
  scratch = [
      pltpu.VMEM((NP,), jnp.float32),
      pltpu.VMEM((NP,), jnp.float32),
      pltpu.VMEM((DEC_NB, BT), jnp.int32),
      pltpu.VMEM((DEC_NB, BT), jnp.int32),
      pltpu.VMEM((DEC_NB, BT), jnp.float32),
  ]
  fn = pl.kernel(body, out_type=out_type, mesh=mesh, scratch_types=scratch,
                 compiler_params=pltpu.CompilerParams(
                     needs_layout_passes=False))
  return fn(p_hbm, q_hbm, ps, pd, ns, nd)


def _att_mat(att_s, att_d, heads, hid):
  cols = []
  for h in range(heads):
    col = [jnp.zeros((hid,), jnp.float32)] * heads
    col[h] = att_s[h]
    cols.append(jnp.concatenate(col))
  for h in range(heads):
    col = [jnp.zeros((hid,), jnp.float32)] * heads
    col[h] = att_d[h]
    cols.append(jnp.concatenate(col))
  while len(cols) < 8:
    cols.append(jnp.zeros((heads * hid,), jnp.float32))
  return jnp.stack(cols, axis=1)


def _split_tables(att, heads):
  asrc = jnp.concatenate(
      [att[:N_NODES, :heads].T,
       jnp.zeros((heads, NP - N_NODES), jnp.float32)], axis=1)
  adst = jnp.concatenate(
      [att[:N_NODES, heads:2 * heads].T,
       jnp.full((heads, NP - N_NODES), -1e30, jnp.float32)], axis=1)
  return asrc, adst


def _pad_dec(ei):
  pad = E_DEC_PAD - N_EDGES
  s = jnp.concatenate([ei[0], jnp.zeros((pad,), jnp.int32)])
  d = jnp.concatenate([ei[1], jnp.zeros((pad,), jnp.int32)])
  return s.reshape(DEC_ROWS, BT), d.reshape(DEC_ROWS, BT)


def kernel(x, edge_index, edge_weight, pos_edge_index, neg_edge_index,
           W1, a1s, a1d, b1, W2, a2s, a2d, b2, Wl, bl):
  del edge_weight
  loops = jnp.arange(N_NODES, dtype=jnp.int32)
  pad = E_PAD - (N_EDGES + N_NODES)
  src = jnp.concatenate([edge_index[0], loops, jnp.zeros((pad,), jnp.int32)])
  dst = jnp.concatenate([edge_index[1], loops,
                         jnp.full((pad,), DUMMY, jnp.int32)])
  src2 = src.reshape(NBT, BT)
  dst2 = dst.reshape(NBT, BT)

  A1 = _att_mat(a1s, a1d, HEADS, HID)
  *h1s, att1 = _tc_layer1(x, W1, A1)
  asrc1, adst1 = _split_tables(att1, HEADS)
  g1 = _sc_gat(src2, dst2, asrc1, adst1, h1s)

  A2 = _att_mat(a2s, a2d, 1, HID)
  *h2s, att2 = _tc_layer2(g1, b1.reshape(1, -1), W2, A2)
  asrc2, adst2 = _split_tables(att2, 1)
  g2 = _sc_gat(src2, dst2, asrc2, adst2, h2s)

  p_t, q_t = _tc_decode_tables(
      g2, b2.reshape(1, -1), Wl[:HID, 0].reshape(1, -1),
      Wl[HID:, 0].reshape(1, -1), bl)
  ps, pd = _pad_dec(pos_edge_index)
  ns, nd = _pad_dec(neg_edge_index)
  pos_o, neg_o = _sc_decode(p_t, q_t, ps, pd, ns, nd)
  return (pos_o.reshape(-1)[:N_EDGES], neg_o.reshape(-1)[:N_EDGES])


# speedup vs baseline: 14.8368x; 1.6765x over previous
import functools

import jax
import jax.numpy as jnp
from jax import lax
from jax.experimental import pallas as pl
from jax.experimental.pallas import tpu as pltpu
from jax.experimental.pallas import tpu_sc as plsc

N_NODES = 10000
N_EDGES = 160000
F_IN = 256
HEADS = 2
HID = 256

NP = 10240
DUMMY = 10001
LANE = 16
NSUB = 16
NCORE = 2
BT = 128
CW = 64
NBW = 88
E_C = NBW * BT
E_PAD = NSUB * E_C
NBT = E_PAD // BT
STRIPE = NP // NSUB

DEC_ROWS = 1280
DEC_NB = DEC_ROWS // 32
E_DEC_PAD = DEC_ROWS * BT


def _k1_body(x_ref, w_ref, a_ref, *outs):
  h = jnp.dot(x_ref[...], w_ref[...], preferred_element_type=jnp.float32)
  outs[-1][...] = jnp.dot(h, a_ref[...], preferred_element_type=jnp.float32)
  for i, o in enumerate(outs[:-1]):
    o[...] = h[:, i * CW:(i + 1) * CW]


def _tc_layer1(x, W1, A1):
  br = 1000
  grid = (N_NODES // br,)
  hs = [jax.ShapeDtypeStruct((N_NODES, CW), jnp.float32) for _ in range(8)]
  return pl.pallas_call(
      _k1_body,
      grid=grid,
      in_specs=[
          pl.BlockSpec((br, F_IN), lambda i: (i, 0)),
          pl.BlockSpec((F_IN, 2 * HID), lambda i: (0, 0)),
          pl.BlockSpec((2 * HID, 8), lambda i: (0, 0)),
      ],
      out_specs=[pl.BlockSpec((br, CW), lambda i: (i, 0)) for _ in range(8)]
      + [pl.BlockSpec((br, 8), lambda i: (i, 0))],
      out_shape=hs + [jax.ShapeDtypeStruct((N_NODES, 8), jnp.float32)],
  )(x, W1, A1)


def _k2_body(*refs):
  gs, (b_ref, w_ref, a_ref), outs = refs[:8], refs[8:11], refs[11:]
  z = jnp.concatenate([g[...] for g in gs], axis=1)
  z = jnp.maximum(z + b_ref[...], 0.0)
  h = jnp.dot(z, w_ref[...], preferred_element_type=jnp.float32)
  outs[-1][...] = jnp.dot(h, a_ref[...], preferred_element_type=jnp.float32)
  for i, o in enumerate(outs[:-1]):
    o[...] = h[:, i * CW:(i + 1) * CW]


def _tc_layer2(gs, b1, W2, A2):
  br = 1024
  grid = (NP // br,)
  hs = [jax.ShapeDtypeStruct((NP, CW), jnp.float32) for _ in range(4)]
  return pl.pallas_call(
      _k2_body,
      grid=grid,
      in_specs=[pl.BlockSpec((br, CW), lambda i: (i, 0)) for _ in range(8)]
      + [
          pl.BlockSpec((1, 2 * HID), lambda i: (0, 0)),
          pl.BlockSpec((2 * HID, HID), lambda i: (0, 0)),
          pl.BlockSpec((HID, 8), lambda i: (0, 0)),
      ],
      out_specs=[pl.BlockSpec((br, CW), lambda i: (i, 0)) for _ in range(4)]
      + [pl.BlockSpec((br, 8), lambda i: (i, 0))],
      out_shape=hs + [jax.ShapeDtypeStruct((NP, 8), jnp.float32)],
  )(*gs, b1, W2, A2)


def _k3_body(g0, g1, g2, g3, b_ref, wa_ref, wb_ref, bl_ref, p_ref, q_ref):
  z = jnp.concatenate([g0[...], g1[...], g2[...], g3[...]], axis=1) + b_ref[...]
  p_ref[...] = jnp.sum(z * wa_ref[...], axis=1) + bl_ref[0]
  q_ref[...] = jnp.sum(z * wb_ref[...], axis=1)


def _tc_decode_tables(gs, b2, wa, wb, bl):
  br = 1024
  grid = (NP // br,)
  return pl.pallas_call(
      _k3_body,
      grid=grid,
      in_specs=[pl.BlockSpec((br, CW), lambda i: (i, 0)) for _ in range(4)]
      + [
          pl.BlockSpec((1, HID), lambda i: (0, 0)),
          pl.BlockSpec((1, HID), lambda i: (0, 0)),
          pl.BlockSpec((1, HID), lambda i: (0, 0)),
          pl.BlockSpec(memory_space=pltpu.SMEM),
      ],
      out_specs=[pl.BlockSpec((br,), lambda i: (i,)) for _ in range(2)],
      out_shape=[jax.ShapeDtypeStruct((NP,), jnp.float32) for _ in range(2)],
  )(*gs, b2, wa, wb, bl)


def _sc_gat(src2, dst2, asrc, adst, h_chunks):
  nch = len(h_chunks)
  pc = nch // NCORE
  mesh = plsc.VectorSubcoreMesh(core_axis_name="c", subcore_axis_name="s",
                                num_cores=NCORE, num_subcores=NSUB)

  def body(src_hbm, dst_hbm, asrc_hbm, adst_hbm, *rest):
    h_refs = rest[:nch]
    out_refs = rest[nch : 2 * nch]
    (src_v, dst_v, coef_v, asrc_t, adst_t, denom_t, rows_v, rows_b, zero1_v,
     sem, sem_b, acc_s, denom_s) = rest[2 * nch :]
    c = lax.axis_index("c")
    s = lax.axis_index("s")
    hd = c if nch == 8 else 0

    pltpu.sync_copy(src_hbm.at[pl.ds(s * NBW, NBW)], src_v)
    pltpu.sync_copy(dst_hbm.at[pl.ds(s * NBW, NBW)], dst_v)
    pltpu.sync_copy(asrc_hbm.at[hd], asrc_t)
    pltpu.sync_copy(adst_hbm.at[hd], adst_t)

    neg = jnp.full((LANE,), -1e30, jnp.float32)

    def _mx(i, carry):
      ms, md = carry
      sl = pl.ds(i * LANE, LANE)
      return jnp.maximum(ms, asrc_t[sl]), jnp.maximum(md, adst_t[sl])

    ms, md = lax.fori_loop(0, NP // LANE, _mx, (neg, neg))
    ms_s, md_s = ms[0], md[0]
    for i in range(1, LANE):
      ms_s = jnp.maximum(ms_s, ms[i])
      md_s = jnp.maximum(md_s, md[i])
    smax = ms_s + md_s
    shift = jnp.where(smax > 0, smax, 0.2 * smax)

    z16 = jnp.zeros((LANE,), jnp.float32)

    @pl.loop(0, STRIPE // LANE)
    def _zz(i):
      zero1_v[pl.ds(i * LANE, LANE)] = z16

    @pl.loop(0, BT)
    def _zr(r):
      for i2 in range(CW // LANE):
        rows_v[r, pl.ds(i2 * LANE, LANE)] = z16

    pltpu.sync_copy(zero1_v, denom_s.at[pl.ds(s * STRIPE, STRIPE)])
    plsc.subcore_barrier()

    @pl.loop(0, NBW)
    def _p1(j):
      for k in range(8):
        sl = pl.ds(k * LANE, LANE)
        sv = src_v[j, sl]
        dv = dst_v[j, sl]
        al = plsc.load_gather(asrc_t, [sv]) + plsc.load_gather(adst_t, [dv])
        al = jnp.where(al > 0, al, 0.2 * al)
        coef_v[j, sl] = jnp.exp(al - shift)
      pltpu.sync_copy(coef_v.at[j], denom_s.at[dst_v.at[j]], add=True)

    plsc.subcore_barrier()

    pltpu.sync_copy(denom_s, denom_t)

    @pl.loop(0, NBW)
    def _p2(j):
      for k in range(8):
        sl = pl.ds(k * LANE, LANE)
        dv = dst_v[j, sl]
        den = plsc.load_gather(denom_t, [dv])
        coef_v[j, sl] = coef_v[j, sl] / (den + 1e-16)

    def agg(tab):
      bufs = (rows_v, rows_b)
      sems = (sem, sem_b)

      def fire(jj, b):
        pltpu.async_copy(tab.at[src_v.at[jj]], bufs[b], sems[b])

      def step(j, b):
        buf = bufs[b]
        pltpu.make_async_copy(tab.at[src_v.at[j]], buf, sems[b]).wait()

        @pl.loop(0, BT // LANE)
        def _sc(k):
          cv = coef_v[j, pl.ds(k * LANE, LANE)]
          for rr in range(LANE):
            cf = cv[rr]
            r = k * LANE + rr
            for i2 in range(CW // LANE):
              sl = pl.ds(i2 * LANE, LANE)
              buf[r, sl] = buf[r, sl] * cf

        pltpu.sync_copy(buf, acc_s.at[dst_v.at[j]], add=True)

        @pl.when(j + 2 < NBW)
        def _nx():
          fire(j + 2, b)

      fire(0, 0)
      fire(1, 1)

      @pl.loop(0, NBW, step=2)
      def _p3(j):
        step(j, 0)
        step(j + 1, 1)

    for p in range(pc):
      if p > 0:
        @pl.loop(0, BT)
        def _zr2(r):
          for i2 in range(CW // LANE):
            rows_v[r, pl.ds(i2 * LANE, LANE)] = z16

      for zz in range(STRIPE // BT):
        pltpu.sync_copy(rows_v, acc_s.at[pl.ds(s * STRIPE + zz * BT, BT)])
      plsc.subcore_barrier()

      @pl.when(c == 0)
      def _c0():
        agg(h_refs[p])

      @pl.when(c == 1)
      def _c1():
        agg(h_refs[pc + p])

      plsc.subcore_barrier()

      @pl.when(c == 0)
      def _o0():
        pltpu.sync_copy(acc_s.at[pl.ds(s * STRIPE, STRIPE)],
                        out_refs[p].at[pl.ds(s * STRIPE, STRIPE)])

      @pl.when(c == 1)
      def _o1():
        pltpu.sync_copy(acc_s.at[pl.ds(s * STRIPE, STRIPE)],
                        out_refs[pc + p].at[pl.ds(s * STRIPE, STRIPE)])

      if p + 1 < pc:
        plsc.subcore_barrier()

  out_type = [jax.ShapeDtypeStruct((NP, CW), jnp.float32) for _ in range(nch)]
  scratch = [
      pltpu.VMEM((NBW, BT), jnp.int32),
      pltpu.VMEM((NBW, BT), jnp.int32),
      pltpu.VMEM((NBW, BT), jnp.float32),
      pltpu.VMEM((NP,), jnp.float32),
      pltpu.VMEM((NP,), jnp.float32),
      pltpu.VMEM((NP,), jnp.float32),
      pltpu.VMEM((BT, CW), jnp.float32),
      pltpu.VMEM((BT, CW), jnp.float32),
      pltpu.VMEM((STRIPE,), jnp.float32),
      pltpu.SemaphoreType.DMA,
      pltpu.SemaphoreType.DMA,
      pltpu.VMEM_SHARED((NP, CW), jnp.float32),
      pltpu.VMEM_SHARED((NP,), jnp.float32),
  ]
  fn = pl.kernel(body, out_type=out_type, mesh=mesh, scratch_types=scratch,
                 compiler_params=pltpu.CompilerParams(
                     needs_layout_passes=False, use_tc_tiling_on_sc=False))
  return fn(src2, dst2, asrc, adst, *h_chunks)


def _sc_decode(p_hbm, q_hbm, ps, pd, ns, nd):
  mesh = plsc.VectorSubcoreMesh(core_axis_name="c", subcore_axis_name="s",
                                num_cores=NCORE, num_subcores=NSUB)

  def body(p_in, q_in, ps_in, pd_in, ns_in, nd_in, pos_out, neg_out,
           p_t, q_t, idx_s, idx_d, out_v):
    c = lax.axis_index("c")
    s = lax.axis_index("s")
    w = s * NCORE + c
    pltpu.sync_copy(p_in, p_t)
    pltpu.sync_copy(q_in, q_t)
    for src_h, dst_h, out_h in ((ps_in, pd_in, pos_out),
                                (ns_in, nd_in, neg_out)):
      pltpu.sync_copy(src_h.at[pl.ds(w * DEC_NB, DEC_NB)], idx_s)
      pltpu.sync_copy(dst_h.at[pl.ds(w * DEC_NB, DEC_NB)], idx_d)

      @pl.loop(0, DEC_NB)
      def _dj(j):
        for k in range(8):
          sl = pl.ds(k * LANE, LANE)
          sv = idx_s[j, sl]
          dv = idx_d[j, sl]
          out_v[j, sl] = (plsc.load_gather(p_t, [sv])
                          + plsc.load_gather(q_t, [dv]))

      pltpu.sync_copy(out_v, out_h.at[pl.ds(w * DEC_NB, DEC_NB)])

  out_type = [jax.ShapeDtypeStruct((DEC_ROWS, BT), jnp.float32)
              for _ in range(2)]
  scratch = [
      pltpu.VMEM((NP,), jnp.float32),
      pltpu.VMEM((NP,), jnp.float32),
      pltpu.VMEM((DEC_NB, BT), jnp.int32),
      pltpu.VMEM((DEC_NB, BT), jnp.int32),
      pltpu.VMEM((DEC_NB, BT), jnp.float32),
  ]
  fn = pl.kernel(body, out_type=out_type, mesh=mesh, scratch_types=scratch,
                 compiler_params=pltpu.CompilerParams(
                     needs_layout_passes=False))
  return fn(p_hbm, q_hbm, ps, pd, ns, nd)


def _att_mat(att_s, att_d, heads, hid):
  cols = []
  for h in range(heads):
    col = [jnp.zeros((hid,), jnp.float32)] * heads
    col[h] = att_s[h]
    cols.append(jnp.concatenate(col))
  for h in range(heads):
    col = [jnp.zeros((hid,), jnp.float32)] * heads
    col[h] = att_d[h]
    cols.append(jnp.concatenate(col))
  while len(cols) < 8:
    cols.append(jnp.zeros((heads * hid,), jnp.float32))
  return jnp.stack(cols, axis=1)


def _split_tables(att, heads):
  asrc = jnp.concatenate(
      [att[:N_NODES, :heads].T,
       jnp.zeros((heads, NP - N_NODES), jnp.float32)], axis=1)
  adst = jnp.concatenate(
      [att[:N_NODES, heads:2 * heads].T,
       jnp.full((heads, NP - N_NODES), -1e30, jnp.float32)], axis=1)
  return asrc, adst


def _pad_dec(ei):
  pad = E_DEC_PAD - N_EDGES
  s = jnp.concatenate([ei[0], jnp.zeros((pad,), jnp.int32)])
  d = jnp.concatenate([ei[1], jnp.zeros((pad,), jnp.int32)])
  return s.reshape(DEC_ROWS, BT), d.reshape(DEC_ROWS, BT)


def kernel(x, edge_index, edge_weight, pos_edge_index, neg_edge_index,
           W1, a1s, a1d, b1, W2, a2s, a2d, b2, Wl, bl):
  del edge_weight
  loops = jnp.arange(N_NODES, dtype=jnp.int32)
  pad = E_PAD - (N_EDGES + N_NODES)
  src = jnp.concatenate([edge_index[0], loops, jnp.zeros((pad,), jnp.int32)])
  dst = jnp.concatenate([edge_index[1], loops,
                         jnp.full((pad,), DUMMY, jnp.int32)])
  src2 = src.reshape(NBT, BT)
  dst2 = dst.reshape(NBT, BT)

  A1 = _att_mat(a1s, a1d, HEADS, HID)
  *h1s, att1 = _tc_layer1(x, W1, A1)
  asrc1, adst1 = _split_tables(att1, HEADS)
  g1 = _sc_gat(src2, dst2, asrc1, adst1, h1s)

  A2 = _att_mat(a2s, a2d, 1, HID)
  *h2s, att2 = _tc_layer2(g1, b1.reshape(1, -1), W2, A2)
  asrc2, adst2 = _split_tables(att2, 1)
  g2 = _sc_gat(src2, dst2, asrc2, adst2, h2s)

  p_t, q_t = _tc_decode_tables(
      g2, b2.reshape(1, -1), Wl[:HID, 0].reshape(1, -1),
      Wl[HID:, 0].reshape(1, -1), bl)
  ps, pd = _pad_dec(pos_edge_index)
  ns, nd = _pad_dec(neg_edge_index)
  pos_o, neg_o = _sc_decode(p_t, q_t, ps, pd, ns, nd)
  return (pos_o.reshape(-1)[:N_EDGES], neg_o.reshape(-1)[:N_EDGES])
